# initial kernel scaffold (unmeasured)
import jax
import jax.numpy as jnp
from jax import lax
from jax.experimental import pallas as pl
from jax.experimental.pallas import tpu as pltpu

N_DEV = 4
SQ = 1024
SKV = 1024
D_MODEL = 1024
H_PER = 8
DH = 128
WINDOW = 128
SCALE = 0.08838834764831843
NEG_INF = -1e9


def _body(x_ref, wq_ref, k_ref, v_ref, wo_ref, out_ref,
          comm_ref, send_sems, recv_sems, exit_sem):
    my = lax.axis_index("i")
    left = lax.rem(my + N_DEV - 1, N_DEV)
    right = lax.rem(my + 1, N_DEV)

    barrier_sem = pltpu.get_barrier_semaphore()
    for nbr in (left, right):
        pl.semaphore_signal(barrier_sem, inc=1, device_id=(nbr,),
                            device_id_type=pl.DeviceIdType.MESH)
    pl.semaphore_wait(barrier_sem, 2)

    comm_ref[0, 0] = wq_ref[...]
    comm_ref[0, 1] = wo_ref[...]

    x_val = x_ref[...]

    row = lax.broadcasted_iota(jnp.int32, (SQ, SKV), 0)
    col = lax.broadcasted_iota(jnp.int32, (SQ, SKV), 1)
    band = jnp.abs(row - col) <= WINDOW

    def compute_origin(slot, g):
        wq = comm_ref[slot, 0]
        wo = comm_ref[slot, 1]
        q = jnp.dot(x_val, wq, preferred_element_type=jnp.float32)
        ks = k_ref[pl.ds(g * H_PER, H_PER)]
        vs = v_ref[pl.ds(g * H_PER, H_PER)]
        ctxs = []
        for h in range(H_PER):
            qh = q[:, h * DH:(h + 1) * DH]
            s = lax.dot_general(qh, ks[h], (((1,), (1,)), ((), ())),
                                preferred_element_type=jnp.float32) * SCALE
            s = jnp.where(band, s, NEG_INF)
            m = jnp.max(s, axis=-1, keepdims=True)
            w = jnp.exp(s - m)
            w = w / jnp.sum(w, axis=-1, keepdims=True)
            ctxs.append(jnp.dot(w, vs[h], preferred_element_type=jnp.float32))
        ctx = jnp.concatenate(ctxs, axis=1)
        return jnp.dot(ctx, wo, preferred_element_type=jnp.float32)

    rdmas = []
    for h in range(N_DEV - 1):
        rdmas.append(pltpu.make_async_remote_copy(
            src_ref=comm_ref.at[h],
            dst_ref=comm_ref.at[h + 1],
            send_sem=send_sems.at[h],
            recv_sem=recv_sems.at[h],
            device_id=(right,),
            device_id_type=pl.DeviceIdType.MESH,
        ))

    rdmas[0].start()
    out_ref[...] = compute_origin(0, my)
    rdmas[0].wait_recv()
    rdmas[1].start()
    out_ref[...] += compute_origin(1, lax.rem(my + 3, N_DEV))
    rdmas[1].wait_recv()
    rdmas[2].start()
    out_ref[...] += compute_origin(2, lax.rem(my + 2, N_DEV))
    rdmas[2].wait_recv()
    out_ref[...] += compute_origin(3, right)
    for r in rdmas:
        r.wait_send()

    for nbr in (left, right):
        pl.semaphore_signal(exit_sem, inc=1, device_id=(nbr,),
                            device_id_type=pl.DeviceIdType.MESH)
    pl.semaphore_wait(exit_sem, 2)


def kernel(x, Wq, K_ext, V_ext, Wo):
    my = lax.axis_index("i")
    x2 = x[0]
    k_loc = jnp.transpose(lax.dynamic_index_in_dim(K_ext, my, 0, False),
                          (1, 0, 2))
    v_loc = jnp.transpose(lax.dynamic_index_in_dim(V_ext, my, 0, False),
                          (1, 0, 2))

    out = pl.pallas_call(
        _body,
        out_shape=jax.ShapeDtypeStruct((SQ, D_MODEL), jnp.float32),
        in_specs=[pl.BlockSpec(memory_space=pltpu.VMEM)] * 5,
        out_specs=pl.BlockSpec(memory_space=pltpu.VMEM),
        scratch_shapes=[
            pltpu.VMEM((N_DEV, 2, D_MODEL, D_MODEL), jnp.float32),
            pltpu.SemaphoreType.DMA((N_DEV - 1,)),
            pltpu.SemaphoreType.DMA((N_DEV - 1,)),
            pltpu.SemaphoreType.REGULAR,
        ],
        compiler_params=pltpu.CompilerParams(collective_id=0),
    )(x2, Wq, k_loc, v_loc, Wo)
    return out[None]


# baseline (device time: 362837 ns/iter reference)
import jax
import jax.numpy as jnp
from jax import lax
from jax.experimental import pallas as pl
from jax.experimental.pallas import tpu as pltpu

N_DEV = 4
SQ = 1024
SKV = 1024
D_MODEL = 1024
H_PER = 8
DH = 128
WINDOW = 128
SCALE = 0.08838834764831843
NEG_INF = -1e9


QB = 256
KW = 512


def _body(x_ref, wq_ref, k_hbm, v_hbm, wo_ref, out_ref,
          comm_ref, k_buf, v_buf, ctx_buf, send_sems, recv_sems,
          kv_sems, credit_sem, exit_sem):
    my = lax.axis_index("i")
    left = lax.rem(my + N_DEV - 1, N_DEV)
    right = lax.rem(my + 1, N_DEV)

    barrier_sem = pltpu.get_barrier_semaphore()
    for nbr in (left, right):
        pl.semaphore_signal(barrier_sem, inc=1, device_id=(nbr,),
                            device_id_type=pl.DeviceIdType.MESH)
    pl.semaphore_wait(barrier_sem, 2)

    def fetch_kv(g):
        ck = pltpu.make_async_copy(
            k_hbm.at[pl.ds(g * H_PER, H_PER)], k_buf, kv_sems.at[0])
        cv = pltpu.make_async_copy(
            v_hbm.at[pl.ds(g * H_PER, H_PER)], v_buf, kv_sems.at[1])
        ck.start()
        cv.start()
        ck.wait()
        cv.wait()

    def compute_origin(wq_col, wo_full, g):
        for h in range(H_PER):
            qh = jnp.dot(x_ref[...], wq_col(h),
                         preferred_element_type=jnp.float32)
            for qb in range(SQ // QB):
                r0 = qb * QB
                w0 = min(max(r0 - WINDOW, 0), SKV - KW)
                qblk = qh[r0:r0 + QB]
                kwin = k_buf[h, w0:w0 + KW]
                s = lax.dot_general(qblk, kwin, (((1,), (1,)), ((), ())),
                                    preferred_element_type=jnp.float32)
                s = s * SCALE
                qi = lax.broadcasted_iota(jnp.int32, (QB, KW), 0)
                ki = lax.broadcasted_iota(jnp.int32, (QB, KW), 1)
                band = jnp.abs(qi - ki + (r0 - w0)) <= WINDOW
                s = jnp.where(band, s, NEG_INF)
                m = jnp.max(s, axis=-1, keepdims=True)
                w = jnp.exp(s - m)
                w = w / jnp.sum(w, axis=-1, keepdims=True)
                ctx_buf[r0:r0 + QB, h * DH:(h + 1) * DH] = jnp.dot(
                    w, v_buf[h, w0:w0 + KW],
                    preferred_element_type=jnp.float32)
        return jnp.dot(ctx_buf[...], wo_full(),
                       preferred_element_type=jnp.float32)

    rdma0a = pltpu.make_async_remote_copy(
        src_ref=wq_ref, dst_ref=comm_ref.at[0, 0],
        send_sem=send_sems.at[0], recv_sem=recv_sems.at[0],
        device_id=(right,), device_id_type=pl.DeviceIdType.MESH)
    rdma0b = pltpu.make_async_remote_copy(
        src_ref=wo_ref, dst_ref=comm_ref.at[0, 1],
        send_sem=send_sems.at[1], recv_sem=recv_sems.at[1],
        device_id=(right,), device_id_type=pl.DeviceIdType.MESH)
    rdma1 = pltpu.make_async_remote_copy(
        src_ref=comm_ref.at[0], dst_ref=comm_ref.at[1],
        send_sem=send_sems.at[2], recv_sem=recv_sems.at[2],
        device_id=(right,), device_id_type=pl.DeviceIdType.MESH)
    rdma2 = pltpu.make_async_remote_copy(
        src_ref=comm_ref.at[1], dst_ref=comm_ref.at[0],
        send_sem=send_sems.at[3], recv_sem=recv_sems.at[3],
        device_id=(right,), device_id_type=pl.DeviceIdType.MESH)

    def slot_readers(slot):
        return (lambda h: comm_ref[slot, 0, :, h * DH:(h + 1) * DH],
                lambda: comm_ref[slot, 1])

    rdma0a.start()
    rdma0b.start()
    fetch_kv(my)
    out_ref[...] = compute_origin(
        lambda h: wq_ref[:, h * DH:(h + 1) * DH], lambda: wo_ref[...], my)

    rdma0a.wait_recv()
    rdma0b.wait_recv()
    rdma1.start()
    fetch_kv(left)
    out_ref[...] += compute_origin(*slot_readers(0), left)
    rdma1.wait_send()
    pl.semaphore_signal(credit_sem, inc=1, device_id=(left,),
                        device_id_type=pl.DeviceIdType.MESH)

    rdma1.wait_recv()
    pl.semaphore_wait(credit_sem, 1)
    rdma2.start()
    g2 = lax.rem(my + 2, N_DEV)
    fetch_kv(g2)
    out_ref[...] += compute_origin(*slot_readers(1), g2)

    rdma2.wait_recv()
    fetch_kv(right)
    out_ref[...] += compute_origin(*slot_readers(0), right)

    rdma0a.wait_send()
    rdma0b.wait_send()
    rdma2.wait_send()

    for nbr in (left, right):
        pl.semaphore_signal(exit_sem, inc=1, device_id=(nbr,),
                            device_id_type=pl.DeviceIdType.MESH)
    pl.semaphore_wait(exit_sem, 2)


def kernel(x, Wq, K_ext, V_ext, Wo):
    my = lax.axis_index("i")
    x2 = x[0]
    k_loc = jnp.transpose(lax.dynamic_index_in_dim(K_ext, my, 0, False),
                          (1, 0, 2))
    v_loc = jnp.transpose(lax.dynamic_index_in_dim(V_ext, my, 0, False),
                          (1, 0, 2))

    out = pl.pallas_call(
        _body,
        out_shape=jax.ShapeDtypeStruct((SQ, D_MODEL), jnp.float32),
        in_specs=[
            pl.BlockSpec(memory_space=pltpu.VMEM),
            pl.BlockSpec(memory_space=pltpu.VMEM),
            pl.BlockSpec(memory_space=pltpu.HBM),
            pl.BlockSpec(memory_space=pltpu.HBM),
            pl.BlockSpec(memory_space=pltpu.VMEM),
        ],
        out_specs=pl.BlockSpec(memory_space=pltpu.VMEM),
        scratch_shapes=[
            pltpu.VMEM((2, 2, D_MODEL, D_MODEL), jnp.float32),
            pltpu.VMEM((H_PER, SKV, DH), jnp.float32),
            pltpu.VMEM((H_PER, SKV, DH), jnp.float32),
            pltpu.VMEM((SQ, H_PER * DH), jnp.float32),
            pltpu.SemaphoreType.DMA((4,)),
            pltpu.SemaphoreType.DMA((4,)),
            pltpu.SemaphoreType.DMA((2,)),
            pltpu.SemaphoreType.REGULAR,
            pltpu.SemaphoreType.REGULAR,
        ],
        compiler_params=pltpu.CompilerParams(
            collective_id=0, vmem_limit_bytes=40 * 1024 * 1024
        ),
    )(x2, Wq, k_loc, v_loc, Wo)
    return out[None]


# device time: 244483 ns/iter; 1.4841x vs baseline; 1.4841x over previous
import jax
import jax.numpy as jnp
from jax import lax
from jax.experimental import pallas as pl
from jax.experimental.pallas import tpu as pltpu

N_DEV = 4
SQ = 1024
SKV = 1024
D_MODEL = 1024
H_PER = 8
DH = 128
WINDOW = 128
SCALE = 0.08838834764831843
NEG_INF = -1e9
QB = 256
KW = 512


def _body(x_ref, wq_ref, k_hbm, v_hbm, wo_ref, out_ref,
          wq_comm, wo_comm, k_buf, v_buf, stash,
          send_sems, recv_sems, kv_sems, credit_sem, exit_sem):
    my = lax.axis_index("i")
    left = lax.rem(my + N_DEV - 1, N_DEV)
    right = lax.rem(my + 1, N_DEV)

    barrier_sem = pltpu.get_barrier_semaphore()
    for nbr in (left, right):
        pl.semaphore_signal(barrier_sem, inc=1, device_id=(nbr,),
                            device_id_type=pl.DeviceIdType.MESH)
    pl.semaphore_wait(barrier_sem, 2)

    def fetch_kv(g):
        ck = pltpu.make_async_copy(
            k_hbm.at[pl.ds(g * H_PER, H_PER)], k_buf, kv_sems.at[0])
        cv = pltpu.make_async_copy(
            v_hbm.at[pl.ds(g * H_PER, H_PER)], v_buf, kv_sems.at[1])
        ck.start()
        cv.start()
        ck.wait()
        cv.wait()

    def block_geom(qb):
        r0 = qb * QB
        w0 = min(max(r0 - WINDOW, 0), SKV - KW)
        qi = lax.broadcasted_iota(jnp.int32, (QB, KW), 0)
        ki = lax.broadcasted_iota(jnp.int32, (QB, KW), 1)
        band = jnp.abs(qi - ki + (r0 - w0)) <= WINDOW
        return r0, w0, band

    def head_ctx(r0, w0, band, h, wq_col):
        qh = jnp.dot(x_ref[r0:r0 + QB], wq_col(h),
                     preferred_element_type=jnp.float32)
        s = lax.dot_general(
            qh, k_buf[h, w0:w0 + KW], (((1,), (1,)), ((), ())),
            preferred_element_type=jnp.float32) * SCALE
        s = jnp.where(band, s, NEG_INF)
        m = jnp.max(s, axis=-1, keepdims=True)
        w = jnp.exp(s - m)
        w = w / jnp.sum(w, axis=-1, keepdims=True)
        return jnp.dot(w, v_buf[h, w0:w0 + KW],
                       preferred_element_type=jnp.float32)

    def term(wq_col, wo_row, first):
        for qb in range(SQ // QB):
            r0, w0, band = block_geom(qb)
            acc = None
            for h in range(H_PER):
                c = head_ctx(r0, w0, band, h, wq_col)
                p = jnp.dot(c, wo_row(h), preferred_element_type=jnp.float32)
                acc = p if acc is None else acc + p
            if first:
                out_ref[r0:r0 + QB] = acc
            else:
                out_ref[r0:r0 + QB] += acc

    def project_stash(wo_row):
        for qb in range(SQ // QB):
            r0 = qb * QB
            acc = None
            for h in range(H_PER):
                p = jnp.dot(stash[r0:r0 + QB, h * DH:(h + 1) * DH], wo_row(h),
                            preferred_element_type=jnp.float32)
                acc = p if acc is None else acc + p
            out_ref[r0:r0 + QB] += acc

    def hop(i, src, dst, target):
        return pltpu.make_async_remote_copy(
            src_ref=src, dst_ref=dst,
            send_sem=send_sems.at[i], recv_sem=recv_sems.at[i],
            device_id=(target,), device_id_type=pl.DeviceIdType.MESH)

    rq0 = hop(0, wq_ref, wq_comm.at[0], right)
    rq1 = hop(1, wq_comm.at[0], wq_comm.at[1], right)
    rq2 = hop(2, wq_comm.at[1], wq_comm.at[0], right)
    ro0 = hop(3, wo_ref, wo_comm.at[0], left)
    ro1 = hop(4, wo_comm.at[0], wo_comm.at[1], left)
    ro2 = hop(5, wo_comm.at[1], wo_comm.at[2], left)

    def wq_slot(s):
        return lambda h: wq_comm[s, :, h * DH:(h + 1) * DH]

    def wo_slot(s):
        return lambda h: wo_comm[s, h * DH:(h + 1) * DH]

    rq0.start()
    ro0.start()
    fetch_kv(my)
    term(lambda h: wq_ref[:, h * DH:(h + 1) * DH],
         lambda h: wo_ref[h * DH:(h + 1) * DH], first=True)

    rq0.wait_recv()
    ro0.wait_recv()
    rq1.start()
    ro1.start()
    fetch_kv(left)
    for qb in range(SQ // QB):
        r0, w0, band = block_geom(qb)
        for h in range(H_PER):
            stash[r0:r0 + QB, h * DH:(h + 1) * DH] = head_ctx(
                r0, w0, band, h, wq_slot(0))
    rq1.wait_send()
    pl.semaphore_signal(credit_sem, inc=1, device_id=(left,),
                        device_id_type=pl.DeviceIdType.MESH)

    rq1.wait_recv()
    ro1.wait_recv()
    pl.semaphore_wait(credit_sem, 1)
    rq2.start()
    ro2.start()
    fetch_kv(lax.rem(my + 2, N_DEV))
    term(wq_slot(1), wo_slot(1), first=False)

    rq2.wait_recv()
    ro2.wait_recv()
    fetch_kv(right)
    term(wq_slot(0), wo_slot(0), first=False)
    project_stash(wo_slot(2))

    for r in (rq0, rq2, ro0, ro1, ro2):
        r.wait_send()

    for nbr in (left, right):
        pl.semaphore_signal(exit_sem, inc=1, device_id=(nbr,),
                            device_id_type=pl.DeviceIdType.MESH)
    pl.semaphore_wait(exit_sem, 2)


def kernel(x, Wq, K_ext, V_ext, Wo):
    my = lax.axis_index("i")
    x2 = x[0]
    k_loc = jnp.transpose(lax.dynamic_index_in_dim(K_ext, my, 0, False),
                          (1, 0, 2))
    v_loc = jnp.transpose(lax.dynamic_index_in_dim(V_ext, my, 0, False),
                          (1, 0, 2))

    out = pl.pallas_call(
        _body,
        out_shape=jax.ShapeDtypeStruct((SQ, D_MODEL), jnp.float32),
        in_specs=[
            pl.BlockSpec(memory_space=pltpu.VMEM),
            pl.BlockSpec(memory_space=pltpu.VMEM),
            pl.BlockSpec(memory_space=pltpu.HBM),
            pl.BlockSpec(memory_space=pltpu.HBM),
            pl.BlockSpec(memory_space=pltpu.VMEM),
        ],
        out_specs=pl.BlockSpec(memory_space=pltpu.VMEM),
        scratch_shapes=[
            pltpu.VMEM((2, D_MODEL, D_MODEL), jnp.float32),
            pltpu.VMEM((3, D_MODEL, D_MODEL), jnp.float32),
            pltpu.VMEM((H_PER, SKV, DH), jnp.float32),
            pltpu.VMEM((H_PER, SKV, DH), jnp.float32),
            pltpu.VMEM((SQ, H_PER * DH), jnp.float32),
            pltpu.SemaphoreType.DMA((6,)),
            pltpu.SemaphoreType.DMA((6,)),
            pltpu.SemaphoreType.DMA((2,)),
            pltpu.SemaphoreType.REGULAR,
            pltpu.SemaphoreType.REGULAR,
        ],
        compiler_params=pltpu.CompilerParams(
            collective_id=0, vmem_limit_bytes=46 * 1024 * 1024
        ),
    )(x2, Wq, k_loc, v_loc, Wo)
    return out[None]
